# baseline (speedup 1.0000x reference)
import functools

import jax
import jax.numpy as jnp
from jax import lax
from jax.experimental import pallas as pl
from jax.experimental.pallas import tpu as pltpu
from jax.experimental.pallas import tpu_sc as plsc

_N = 10000
_M = 40000
_NNZ = 320000
_D = 128

_NT = 16
_EPT = _NNZ // _NT
_CL = 128
_CH = 160
_PADDED = _CH * _CL
_ZROWS = 2504
_CW = 512
_CN = _PADDED // _CW

_mesh = plsc.VectorSubcoreMesh(core_axis_name="c", subcore_axis_name="s")
_sc_params = pltpu.CompilerParams(use_tc_tiling_on_sc=False)


def _pad_idx(idx, base):
    idx2 = idx.reshape(_NT, _EPT)
    pad = base + (jnp.arange(_PADDED - _EPT, dtype=jnp.int32) % 8)
    pad2 = jnp.broadcast_to(pad, (_NT, _PADDED - _EPT))
    return jnp.concatenate([idx2, pad2], axis=1).reshape(_NT, _CH, _CL)


def _degrees(rows3, cols3, zeros):
    @functools.partial(
        pl.kernel,
        out_type=(
            jax.ShapeDtypeStruct((_N, 16), jnp.float32),
            jax.ShapeDtypeStruct((_M, 16), jnp.float32),
        ),
        mesh=_mesh,
        scratch_types=[
            pltpu.VMEM((_CH, _CL), jnp.int32),
            pltpu.VMEM((_CL, 16), jnp.float32),
            pltpu.VMEM_SHARED((16 * _ZROWS, 16), jnp.float32),
        ],
        compiler_params=_sc_params,
    )
    def k(rows_hbm, cols_hbm, zeros_hbm, dv_hbm, de_hbm, idx_v, ones_v, acc_sh):
        core = lax.axis_index("c")
        s = lax.axis_index("s")

        @pl.loop(0, _CL)
        def _(j):
            ones_v[j, :] = jnp.full((16,), 1.0, jnp.float32)

        pltpu.sync_copy(zeros_hbm,
                        acc_sh.at[pl.ds(s * _ZROWS, _ZROWS)])

        @pl.when(core == 0)
        def _():
            pltpu.sync_copy(rows_hbm.at[s], idx_v)

        @pl.when(core == 1)
        def _():
            pltpu.sync_copy(cols_hbm.at[s], idx_v)

        plsc.subcore_barrier()

        @pl.loop(0, _CH)
        def _(j):
            pltpu.sync_copy(ones_v, acc_sh.at[idx_v.at[j]], add=True)

        plsc.subcore_barrier()

        @pl.when(core == 0)
        def _():
            pltpu.sync_copy(acc_sh.at[pl.ds(s * (_N // _NT), _N // _NT)],
                            dv_hbm.at[pl.ds(s * (_N // _NT), _N // _NT)])

        @pl.when(core == 1)
        def _():
            pltpu.sync_copy(acc_sh.at[pl.ds(s * (_M // _NT), _M // _NT)],
                            de_hbm.at[pl.ds(s * (_M // _NT), _M // _NT)])

    return k(rows3, cols3, zeros)


def _segsum(src, gidx3, sidx3, zeros, n_src, n_out):
    spt = n_src // _NT
    opt = n_out // _NT
    zpt = -(-(n_out + 8) // _NT)
    acc_rows = _NT * zpt

    @functools.partial(
        pl.kernel,
        out_type=jax.ShapeDtypeStruct((n_out, _D), jnp.float32),
        mesh=_mesh,
        scratch_types=[
            pltpu.VMEM((_CN, _CW), jnp.int32),
            pltpu.VMEM((_CN, _CW), jnp.int32),
            pltpu.VMEM((_CW, 16), jnp.float32),
            pltpu.VMEM((_CW, 16), jnp.float32),
            pltpu.VMEM_SHARED((n_src, 16), jnp.float32),
            pltpu.VMEM_SHARED((acc_rows, 16), jnp.float32),
            pltpu.SemaphoreType.DMA,
            pltpu.SemaphoreType.DMA,
            pltpu.SemaphoreType.DMA,
            pltpu.SemaphoreType.DMA,
        ],
        compiler_params=_sc_params,
    )
    def k(src_hbm, g_hbm, s_hbm, z_hbm, out_hbm, gix, six, buf_a, buf_b,
          src_sh, acc_sh, gs_a, gs_b, ss_a, ss_b):
        core = lax.axis_index("c")
        s = lax.axis_index("s")

        pltpu.async_copy(g_hbm.at[s], gix, gs_a)
        pltpu.async_copy(s_hbm.at[s], six, gs_b)

        for sweep in range(4):
            lane = (core * 4 + sweep) * 16
            plsc.subcore_barrier()
            zc = pltpu.async_copy(z_hbm.at[pl.ds(0, zpt)],
                                  acc_sh.at[pl.ds(s * zpt, zpt)], ss_a)
            sc = pltpu.async_copy(
                src_hbm.at[pl.ds(s * spt, spt), pl.ds(lane, 16)],
                src_sh.at[pl.ds(s * spt, spt)], ss_b)
            if sweep == 0:
                pltpu.make_async_copy(g_hbm.at[s], gix, gs_a).wait()
                pltpu.make_async_copy(s_hbm.at[s], six, gs_b).wait()
            zc.wait()
            sc.wait()
            plsc.subcore_barrier()

            pltpu.async_copy(src_sh.at[gix.at[0]], buf_a, gs_a)

            @pl.loop(0, _CN, step=2)
            def _(p):
                @pl.when(p > 0)
                def _():
                    pltpu.make_async_copy(
                        buf_b, acc_sh.at[six.at[p - 1]], ss_b).wait()

                pltpu.async_copy(src_sh.at[gix.at[p + 1]], buf_b, gs_b)
                pltpu.make_async_copy(src_sh.at[gix.at[p]], buf_a, gs_a).wait()
                pltpu.async_copy(buf_a, acc_sh.at[six.at[p]], ss_a, add=True)
                pltpu.make_async_copy(
                    buf_a, acc_sh.at[six.at[p]], ss_a).wait()

                @pl.when(p + 2 < _CN)
                def _():
                    pltpu.async_copy(src_sh.at[gix.at[p + 2]], buf_a, gs_a)

                pltpu.make_async_copy(
                    src_sh.at[gix.at[p + 1]], buf_b, gs_b).wait()
                pltpu.async_copy(buf_b, acc_sh.at[six.at[p + 1]], ss_b,
                                 add=True)

            pltpu.make_async_copy(
                buf_b, acc_sh.at[six.at[_CN - 1]], ss_b).wait()

            plsc.subcore_barrier()
            pltpu.sync_copy(acc_sh.at[pl.ds(s * opt, opt)],
                            out_hbm.at[pl.ds(s * opt, opt), pl.ds(lane, 16)])

    return k(src, gidx3, sidx3, zeros)


def _linear(x, W, b):
    def body(x_ref, w_ref, b_ref, o_ref):
        o_ref[...] = (
            jnp.dot(x_ref[...], w_ref[...].T, preferred_element_type=jnp.float32)
            + b_ref[...]
        )

    return pl.pallas_call(
        body,
        grid=(10,),
        out_shape=jax.ShapeDtypeStruct((_N, _D), jnp.float32),
        in_specs=[
            pl.BlockSpec((_N // 10, _D), lambda i: (i, 0)),
            pl.BlockSpec((_D, _D), lambda i: (0, 0)),
            pl.BlockSpec((_D,), lambda i: (0,)),
        ],
        out_specs=pl.BlockSpec((_N // 10, _D), lambda i: (i, 0)),
    )(x, W, b)


def _dv_inv_sqrt(dv16_blk):
    d = dv16_blk[:, 0:1]
    return jnp.where(d > 0, 1.0 / jnp.sqrt(jnp.maximum(d, 1e-12)), 0.0)


def _de_inv(de16_blk):
    d = de16_blk[:, 0:1]
    return jnp.where(d > 0, 1.0 / jnp.maximum(d, 1e-12), 0.0)


def _scale_h(h_raw, dv16):
    def body(h_ref, d_ref, o_ref):
        o_ref[...] = h_ref[...] * _dv_inv_sqrt(d_ref[...])

    return pl.pallas_call(
        body,
        grid=(5,),
        out_shape=jax.ShapeDtypeStruct((_N, _D), jnp.float32),
        in_specs=[
            pl.BlockSpec((_N // 5, _D), lambda i: (i, 0)),
            pl.BlockSpec((_N // 5, 16), lambda i: (i, 0)),
        ],
        out_specs=pl.BlockSpec((_N // 5, _D), lambda i: (i, 0)),
    )(h_raw, dv16)


def _scale_t(t_raw, de16):
    def body(t_ref, d_ref, o_ref):
        o_ref[...] = t_ref[...] * _de_inv(d_ref[...])

    return pl.pallas_call(
        body,
        grid=(8,),
        out_shape=jax.ShapeDtypeStruct((_M, _D), jnp.float32),
        in_specs=[
            pl.BlockSpec((_M // 8, _D), lambda i: (i, 0)),
            pl.BlockSpec((_M // 8, 16), lambda i: (i, 0)),
        ],
        out_specs=pl.BlockSpec((_M // 8, _D), lambda i: (i, 0)),
    )(t_raw, de16)


def _finish_e(e_raw, dv16, x):
    def body(e_ref, d_ref, x_ref, e2_ref, node_ref):
        e = e_ref[...] * _dv_inv_sqrt(d_ref[...])
        e = jnp.where(e >= 0, e, 0.01 * e)
        e2_ref[...] = e
        node_ref[...] = (x_ref[...] + e) * 0.5

    return pl.pallas_call(
        body,
        grid=(5,),
        out_shape=(
            jax.ShapeDtypeStruct((_N, _D), jnp.float32),
            jax.ShapeDtypeStruct((_N, _D), jnp.float32),
        ),
        in_specs=[
            pl.BlockSpec((_N // 5, _D), lambda i: (i, 0)),
            pl.BlockSpec((_N // 5, 16), lambda i: (i, 0)),
            pl.BlockSpec((_N // 5, _D), lambda i: (i, 0)),
        ],
        out_specs=(
            pl.BlockSpec((_N // 5, _D), lambda i: (i, 0)),
            pl.BlockSpec((_N // 5, _D), lambda i: (i, 0)),
        ),
    )(e_raw, dv16, x)


def kernel(x, W, b, rows, cols):
    rows_s = _pad_idx(rows, _N)
    cols_s = _pad_idx(cols, _M)
    rows_s9 = rows_s.reshape(_NT, _CN, _CW)
    cols_s9 = cols_s.reshape(_NT, _CN, _CW)
    rows_g = _pad_idx(rows, 0).reshape(_NT, _CN, _CW)
    cols_g = _pad_idx(cols, 0).reshape(_NT, _CN, _CW)
    zeros = jnp.zeros((_ZROWS, 16), jnp.float32)

    dv16, de16 = _degrees(rows_s, cols_s, zeros)
    h_raw = _linear(x, W, b)
    h = _scale_h(h_raw, dv16)

    t_raw = _segsum(h, rows_g, cols_s9, zeros, _N, _M)
    t = _scale_t(t_raw, de16)

    e_raw = _segsum(t, cols_g, rows_s9, zeros, _M, _N)
    e2, node_out = _finish_e(e_raw, dv16, x)

    o_raw = _segsum(e2, rows_g, cols_s9, zeros, _N, _M)
    o = _scale_t(o_raw, de16)

    return (node_out, o)


# speedup vs baseline: 1.0186x; 1.0186x over previous
import functools

import jax
import jax.numpy as jnp
from jax import lax
from jax.experimental import pallas as pl
from jax.experimental.pallas import tpu as pltpu
from jax.experimental.pallas import tpu_sc as plsc

_N = 10000
_M = 40000
_NNZ = 320000
_D = 128

_NT = 16
_EPT = _NNZ // _NT
_CL = 128
_CH = 160
_PADDED = _CH * _CL
_ZROWS = 2504
_CW = 256
_CN = _PADDED // _CW

_mesh = plsc.VectorSubcoreMesh(core_axis_name="c", subcore_axis_name="s")
_sc_params = pltpu.CompilerParams(use_tc_tiling_on_sc=False)


def _pad_idx(idx, base):
    idx2 = idx.reshape(_NT, _EPT)
    pad = base + (jnp.arange(_PADDED - _EPT, dtype=jnp.int32) % 8)
    pad2 = jnp.broadcast_to(pad, (_NT, _PADDED - _EPT))
    return jnp.concatenate([idx2, pad2], axis=1).reshape(_NT, _CH, _CL)


def _degrees(rows3, cols3, zeros):
    @functools.partial(
        pl.kernel,
        out_type=(
            jax.ShapeDtypeStruct((_N, 16), jnp.float32),
            jax.ShapeDtypeStruct((_M, 16), jnp.float32),
        ),
        mesh=_mesh,
        scratch_types=[
            pltpu.VMEM((_CN, _CW), jnp.int32),
            pltpu.VMEM((_CW, 16), jnp.float32),
            pltpu.VMEM_SHARED((16 * _ZROWS, 16), jnp.float32),
            pltpu.SemaphoreType.DMA,
            pltpu.SemaphoreType.DMA,
        ],
        compiler_params=_sc_params,
    )
    def k(rows_hbm, cols_hbm, zeros_hbm, dv_hbm, de_hbm, idx_v, ones_v,
          acc_sh, sem_a, sem_b):
        core = lax.axis_index("c")
        s = lax.axis_index("s")

        @pl.loop(0, _CW)
        def _(j):
            ones_v[j, :] = jnp.full((16,), 1.0, jnp.float32)

        pltpu.sync_copy(zeros_hbm,
                        acc_sh.at[pl.ds(s * _ZROWS, _ZROWS)])

        @pl.when(core == 0)
        def _():
            pltpu.sync_copy(rows_hbm.at[s], idx_v)

        @pl.when(core == 1)
        def _():
            pltpu.sync_copy(cols_hbm.at[s], idx_v)

        plsc.subcore_barrier()

        pltpu.async_copy(ones_v, acc_sh.at[idx_v.at[0]], sem_a, add=True)

        @pl.loop(0, _CN, step=2)
        def _(j):
            pltpu.async_copy(ones_v, acc_sh.at[idx_v.at[j + 1]], sem_b,
                             add=True)
            pltpu.make_async_copy(ones_v, acc_sh.at[idx_v.at[j]], sem_a).wait()

            @pl.when(j + 2 < _CN)
            def _():
                pltpu.async_copy(ones_v, acc_sh.at[idx_v.at[j + 2]], sem_a,
                                 add=True)

            pltpu.make_async_copy(
                ones_v, acc_sh.at[idx_v.at[j + 1]], sem_b).wait()

        plsc.subcore_barrier()

        @pl.when(core == 0)
        def _():
            pltpu.sync_copy(acc_sh.at[pl.ds(s * (_N // _NT), _N // _NT)],
                            dv_hbm.at[pl.ds(s * (_N // _NT), _N // _NT)])

        @pl.when(core == 1)
        def _():
            pltpu.sync_copy(acc_sh.at[pl.ds(s * (_M // _NT), _M // _NT)],
                            de_hbm.at[pl.ds(s * (_M // _NT), _M // _NT)])

    return k(rows3, cols3, zeros)


def _segsum(src, gidx3, sidx3, zeros, n_src, n_out):
    spt = n_src // _NT
    opt = n_out // _NT
    zpt = -(-(n_out + 8) // _NT)
    acc_rows = _NT * zpt

    @functools.partial(
        pl.kernel,
        out_type=jax.ShapeDtypeStruct((n_out, _D), jnp.float32),
        mesh=_mesh,
        scratch_types=[
            pltpu.VMEM((_CN, _CW), jnp.int32),
            pltpu.VMEM((_CN, _CW), jnp.int32),
            pltpu.VMEM((_CW, 16), jnp.float32),
            pltpu.VMEM((_CW, 16), jnp.float32),
            pltpu.VMEM_SHARED((n_src, 16), jnp.float32),
            pltpu.VMEM_SHARED((acc_rows, 16), jnp.float32),
            pltpu.SemaphoreType.DMA,
            pltpu.SemaphoreType.DMA,
            pltpu.SemaphoreType.DMA,
            pltpu.SemaphoreType.DMA,
        ],
        compiler_params=_sc_params,
    )
    def k(src_hbm, g_hbm, s_hbm, z_hbm, out_hbm, gix, six, buf_a, buf_b,
          src_sh, acc_sh, gs_a, gs_b, ss_a, ss_b):
        core = lax.axis_index("c")
        s = lax.axis_index("s")

        pltpu.async_copy(g_hbm.at[s], gix, gs_a)
        pltpu.async_copy(s_hbm.at[s], six, gs_b)

        for sweep in range(4):
            lane = (core * 4 + sweep) * 16
            plsc.subcore_barrier()
            zc = pltpu.async_copy(z_hbm.at[pl.ds(0, zpt)],
                                  acc_sh.at[pl.ds(s * zpt, zpt)], ss_a)
            sc = pltpu.async_copy(
                src_hbm.at[pl.ds(s * spt, spt), pl.ds(lane, 16)],
                src_sh.at[pl.ds(s * spt, spt)], ss_b)
            if sweep == 0:
                pltpu.make_async_copy(g_hbm.at[s], gix, gs_a).wait()
                pltpu.make_async_copy(s_hbm.at[s], six, gs_b).wait()
            zc.wait()
            sc.wait()
            plsc.subcore_barrier()

            pltpu.async_copy(src_sh.at[gix.at[0]], buf_a, gs_a)

            @pl.loop(0, _CN, step=2)
            def _(p):
                @pl.when(p > 0)
                def _():
                    pltpu.make_async_copy(
                        buf_b, acc_sh.at[six.at[p - 1]], ss_b).wait()

                pltpu.async_copy(src_sh.at[gix.at[p + 1]], buf_b, gs_b)
                pltpu.make_async_copy(src_sh.at[gix.at[p]], buf_a, gs_a).wait()
                pltpu.async_copy(buf_a, acc_sh.at[six.at[p]], ss_a, add=True)
                pltpu.make_async_copy(
                    buf_a, acc_sh.at[six.at[p]], ss_a).wait()

                @pl.when(p + 2 < _CN)
                def _():
                    pltpu.async_copy(src_sh.at[gix.at[p + 2]], buf_a, gs_a)

                pltpu.make_async_copy(
                    src_sh.at[gix.at[p + 1]], buf_b, gs_b).wait()
                pltpu.async_copy(buf_b, acc_sh.at[six.at[p + 1]], ss_b,
                                 add=True)

            pltpu.make_async_copy(
                buf_b, acc_sh.at[six.at[_CN - 1]], ss_b).wait()

            plsc.subcore_barrier()
            pltpu.sync_copy(acc_sh.at[pl.ds(s * opt, opt)],
                            out_hbm.at[pl.ds(s * opt, opt), pl.ds(lane, 16)])

    return k(src, gidx3, sidx3, zeros)


def _linear(x, W, b):
    def body(x_ref, w_ref, b_ref, o_ref):
        o_ref[...] = (
            jnp.dot(x_ref[...], w_ref[...].T, preferred_element_type=jnp.float32)
            + b_ref[...]
        )

    return pl.pallas_call(
        body,
        grid=(10,),
        out_shape=jax.ShapeDtypeStruct((_N, _D), jnp.float32),
        in_specs=[
            pl.BlockSpec((_N // 10, _D), lambda i: (i, 0)),
            pl.BlockSpec((_D, _D), lambda i: (0, 0)),
            pl.BlockSpec((_D,), lambda i: (0,)),
        ],
        out_specs=pl.BlockSpec((_N // 10, _D), lambda i: (i, 0)),
    )(x, W, b)


def _dv_inv_sqrt(dv16_blk):
    d = dv16_blk[:, 0:1]
    return jnp.where(d > 0, 1.0 / jnp.sqrt(jnp.maximum(d, 1e-12)), 0.0)


def _de_inv(de16_blk):
    d = de16_blk[:, 0:1]
    return jnp.where(d > 0, 1.0 / jnp.maximum(d, 1e-12), 0.0)


def _scale_h(h_raw, dv16):
    def body(h_ref, d_ref, o_ref):
        o_ref[...] = h_ref[...] * _dv_inv_sqrt(d_ref[...])

    return pl.pallas_call(
        body,
        grid=(5,),
        out_shape=jax.ShapeDtypeStruct((_N, _D), jnp.float32),
        in_specs=[
            pl.BlockSpec((_N // 5, _D), lambda i: (i, 0)),
            pl.BlockSpec((_N // 5, 16), lambda i: (i, 0)),
        ],
        out_specs=pl.BlockSpec((_N // 5, _D), lambda i: (i, 0)),
    )(h_raw, dv16)


def _scale_t(t_raw, de16):
    def body(t_ref, d_ref, o_ref):
        o_ref[...] = t_ref[...] * _de_inv(d_ref[...])

    return pl.pallas_call(
        body,
        grid=(8,),
        out_shape=jax.ShapeDtypeStruct((_M, _D), jnp.float32),
        in_specs=[
            pl.BlockSpec((_M // 8, _D), lambda i: (i, 0)),
            pl.BlockSpec((_M // 8, 16), lambda i: (i, 0)),
        ],
        out_specs=pl.BlockSpec((_M // 8, _D), lambda i: (i, 0)),
    )(t_raw, de16)


def _finish_e(e_raw, dv16, x):
    def body(e_ref, d_ref, x_ref, e2_ref, node_ref):
        e = e_ref[...] * _dv_inv_sqrt(d_ref[...])
        e = jnp.where(e >= 0, e, 0.01 * e)
        e2_ref[...] = e
        node_ref[...] = (x_ref[...] + e) * 0.5

    return pl.pallas_call(
        body,
        grid=(5,),
        out_shape=(
            jax.ShapeDtypeStruct((_N, _D), jnp.float32),
            jax.ShapeDtypeStruct((_N, _D), jnp.float32),
        ),
        in_specs=[
            pl.BlockSpec((_N // 5, _D), lambda i: (i, 0)),
            pl.BlockSpec((_N // 5, 16), lambda i: (i, 0)),
            pl.BlockSpec((_N // 5, _D), lambda i: (i, 0)),
        ],
        out_specs=(
            pl.BlockSpec((_N // 5, _D), lambda i: (i, 0)),
            pl.BlockSpec((_N // 5, _D), lambda i: (i, 0)),
        ),
    )(e_raw, dv16, x)


def kernel(x, W, b, rows, cols):
    rows_s = _pad_idx(rows, _N)
    cols_s = _pad_idx(cols, _M)
    rows_s9 = rows_s.reshape(_NT, _CN, _CW)
    cols_s9 = cols_s.reshape(_NT, _CN, _CW)
    rows_g = _pad_idx(rows, 0).reshape(_NT, _CN, _CW)
    cols_g = _pad_idx(cols, 0).reshape(_NT, _CN, _CW)
    zeros = jnp.zeros((_ZROWS, 16), jnp.float32)

    dv16, de16 = _degrees(rows_s9, cols_s9, zeros)
    h_raw = _linear(x, W, b)
    h = _scale_h(h_raw, dv16)

    t_raw = _segsum(h, rows_g, cols_s9, zeros, _N, _M)
    t = _scale_t(t_raw, de16)

    e_raw = _segsum(t, cols_g, rows_s9, zeros, _M, _N)
    e2, node_out = _finish_e(e_raw, dv16, x)

    o_raw = _segsum(e2, rows_g, cols_s9, zeros, _N, _M)
    o = _scale_t(o_raw, de16)

    return (node_out, o)
